# Initial kernel scaffold; baseline (speedup 1.0000x reference)
import jax
import jax.numpy as jnp
from jax.experimental import pallas as pl


def kernel(x, edge_index, batch, node_imp, W1_0, b1_0, W2_0, b2_0, gamma_0, beta_0, W1_1, b1_1, W2_1, b2_1, gamma_1, beta_1, W1_2, b1_2, W2_2, b2_2, gamma_2, beta_2):
    raise NotImplementedError("write your pallas kernel here")


# speedup vs baseline: 2.7733x; 2.7733x over previous
import functools

import jax
import jax.numpy as jnp
from jax import lax
from jax.experimental import pallas as pl
from jax.experimental.pallas import tpu as pltpu
from jax.experimental.pallas import tpu_sc as plsc

N_NODES = 10000
D_FEAT = 128
G_GRAPHS = 128
N_LAYERS = 3
EPS_GIN = 0.1
EPS_BN = 1e-5

NW = 32
CHUNK = 128
N_PAD = 10240
R_BLK = 2000


@functools.lru_cache(maxsize=None)
def _sc_aggregate(kc: int):
    rows_per_tile = N_PAD // 16
    stage = 8

    @functools.partial(
        pl.kernel,
        out_type=jax.ShapeDtypeStruct((2, N_PAD, D_FEAT), jnp.float32),
        mesh=plsc.VectorSubcoreMesh(core_axis_name="c", subcore_axis_name="s"),
        scratch_types=[
            pltpu.VMEM((stage, CHUNK), jnp.int32),
            pltpu.VMEM((stage, CHUNK), jnp.int32),
            pltpu.VMEM((CHUNK, D_FEAT), jnp.float32),
            pltpu.VMEM_SHARED((N_PAD, D_FEAT), jnp.float32),
            pltpu.SemaphoreType.DMA,
        ],
    )
    def agg(h_hbm, src_hbm, dst_hbm, out_hbm, sidx, didx, rows, acc, sem):
        c = lax.axis_index("c")
        s = lax.axis_index("s")
        wid = c * 16 + s
        zero16 = jnp.zeros((16,), jnp.float32)

        def zrow(i, carry):
            for jj in range(D_FEAT // 16):
                rows[i, pl.ds(jj * 16, 16)] = zero16
            return carry

        lax.fori_loop(0, CHUNK, zrow, 0)
        base = s * rows_per_tile
        for k in range(rows_per_tile // CHUNK):
            pltpu.sync_copy(rows, acc.at[pl.ds(base + k * CHUNK, CHUNK)])
        plsc.subcore_barrier()

        def stage_body(t, carry):
            r0 = wid * kc + t * stage
            pltpu.sync_copy(src_hbm.at[pl.ds(r0, stage)], sidx)
            pltpu.sync_copy(dst_hbm.at[pl.ds(r0, stage)], didx)

            def chunk_body(j, carry2):
                pltpu.async_copy(h_hbm.at[sidx.at[j]], rows, sem).wait()
                pltpu.sync_copy(rows, acc.at[didx.at[j]], add=True)
                return carry2

            lax.fori_loop(0, stage, chunk_body, 0)
            return carry

        lax.fori_loop(0, kc // stage, stage_body, 0)
        plsc.subcore_barrier()

        for k in range(rows_per_tile // CHUNK):
            r0 = base + k * CHUNK
            pltpu.sync_copy(acc.at[pl.ds(r0, CHUNK)], rows)
            pltpu.sync_copy(rows, out_hbm.at[c, pl.ds(r0, CHUNK)])

    return agg


def _aggregate(h, src2, dst2):
    return _sc_aggregate(src2.shape[0] // NW)(h, src2, dst2)


def _segmax_body(nimp_ref, batch_ref, out_ref):
    i = pl.program_id(0)
    gids = lax.broadcasted_iota(jnp.int32, (R_BLK, G_GRAPHS), 1).astype(jnp.float32)
    oh = batch_ref[...] == gids
    m = jnp.max(jnp.where(oh, nimp_ref[...], -jnp.inf), axis=0, keepdims=True)

    @pl.when(i == 0)
    def _():
        out_ref[...] = jnp.full_like(out_ref, -jnp.inf)

    out_ref[0:1, :] = jnp.maximum(out_ref[0:1, :], m)


def _segmax_call(nimp, batch_f):
    return pl.pallas_call(
        _segmax_body,
        grid=(N_NODES // R_BLK,),
        in_specs=[
            pl.BlockSpec((R_BLK, 1), lambda i: (i, 0)),
            pl.BlockSpec((R_BLK, 1), lambda i: (i, 0)),
        ],
        out_specs=pl.BlockSpec((8, G_GRAPHS), lambda i: (0, 0)),
        out_shape=jax.ShapeDtypeStruct((8, G_GRAPHS), jnp.float32),
    )(nimp, batch_f)


def _u1_body(agg_ref, h_ref, w1_ref, b1_ref, w2_ref, b2_ref, t2_ref, sums_ref):
    i = pl.program_id(0)
    t = agg_ref[0] + agg_ref[1] + (1.0 + EPS_GIN) * h_ref[...]
    t = jnp.maximum(
        jnp.dot(t, w1_ref[...], preferred_element_type=jnp.float32) + b1_ref[...],
        0.0,
    )
    t = jnp.dot(t, w2_ref[...], preferred_element_type=jnp.float32) + b2_ref[...]
    t = jnp.maximum(t, 0.0)
    t2_ref[...] = t

    @pl.when(i == 0)
    def _():
        sums_ref[...] = jnp.zeros_like(sums_ref)

    sums_ref[0:1, :] = sums_ref[0:1, :] + jnp.sum(t, axis=0, keepdims=True)
    sums_ref[1:2, :] = sums_ref[1:2, :] + jnp.sum(t * t, axis=0, keepdims=True)


def _u1_call(agg, h, w1, b1, w2, b2):
    return pl.pallas_call(
        _u1_body,
        grid=(N_NODES // R_BLK,),
        in_specs=[
            pl.BlockSpec((2, R_BLK, D_FEAT), lambda i: (0, i, 0)),
            pl.BlockSpec((R_BLK, D_FEAT), lambda i: (i, 0)),
            pl.BlockSpec((D_FEAT, D_FEAT), lambda i: (0, 0)),
            pl.BlockSpec((1, D_FEAT), lambda i: (0, 0)),
            pl.BlockSpec((D_FEAT, D_FEAT), lambda i: (0, 0)),
            pl.BlockSpec((1, D_FEAT), lambda i: (0, 0)),
        ],
        out_specs=[
            pl.BlockSpec((R_BLK, D_FEAT), lambda i: (i, 0)),
            pl.BlockSpec((8, D_FEAT), lambda i: (0, 0)),
        ],
        out_shape=[
            jax.ShapeDtypeStruct((N_NODES, D_FEAT), jnp.float32),
            jax.ShapeDtypeStruct((8, D_FEAT), jnp.float32),
        ],
    )(agg, h, w1, b1, w2, b2)


def _bn_ni(t2_ref, sums_ref, g_ref, b_ref, nimp_ref, batch_ref, segmax_ref):
    s0 = sums_ref[0:1, :]
    s1 = sums_ref[1:2, :]
    mean = s0 / N_NODES
    var = s1 / N_NODES - mean * mean
    scale = g_ref[...] / jnp.sqrt(var + EPS_BN)
    tn = (t2_ref[...] - mean) * scale + b_ref[...]
    gids = lax.broadcasted_iota(jnp.int32, (R_BLK, G_GRAPHS), 1).astype(jnp.float32)
    oh = batch_ref[...] == gids
    smax_row = jnp.sum(jnp.where(oh, segmax_ref[0:1, :], 0.0), axis=1, keepdims=True)
    ni = nimp_ref[...] / (smax_row * 10.0) + 0.9
    return tn, tn * ni, oh


def _u2_body(t2_ref, sums_ref, g_ref, b_ref, nimp_ref, batch_ref, segmax_ref,
             hn_ref, xs_ref):
    tn, xv, _ = _bn_ni(t2_ref, sums_ref, g_ref, b_ref, nimp_ref, batch_ref,
                       segmax_ref)
    hn_ref[...] = tn
    xs_ref[...] = xv


def _u2_pool_body(t2_ref, sums_ref, g_ref, b_ref, nimp_ref, batch_ref,
                  segmax_ref, hn_ref, xs_ref, pool_ref):
    i = pl.program_id(0)
    tn, xv, oh = _bn_ni(t2_ref, sums_ref, g_ref, b_ref, nimp_ref, batch_ref,
                        segmax_ref)
    hn_ref[...] = tn
    xs_ref[...] = xv

    @pl.when(i == 0)
    def _():
        pool_ref[...] = jnp.zeros_like(pool_ref)

    pool_ref[...] = pool_ref[...] + lax.dot_general(
        oh.astype(jnp.float32), xv, (((0,), (0,)), ((), ())),
        preferred_element_type=jnp.float32)


def _u2_call(t2, sums, g, b, nimp, batch_f, segmax, with_pool):
    in_specs = [
        pl.BlockSpec((R_BLK, D_FEAT), lambda i: (i, 0)),
        pl.BlockSpec((8, D_FEAT), lambda i: (0, 0)),
        pl.BlockSpec((1, D_FEAT), lambda i: (0, 0)),
        pl.BlockSpec((1, D_FEAT), lambda i: (0, 0)),
        pl.BlockSpec((R_BLK, 1), lambda i: (i, 0)),
        pl.BlockSpec((R_BLK, 1), lambda i: (i, 0)),
        pl.BlockSpec((8, G_GRAPHS), lambda i: (0, 0)),
    ]
    out_specs = [
        pl.BlockSpec((R_BLK, D_FEAT), lambda i: (i, 0)),
        pl.BlockSpec((R_BLK, D_FEAT), lambda i: (i, 0)),
    ]
    out_shape = [
        jax.ShapeDtypeStruct((N_NODES, D_FEAT), jnp.float32),
        jax.ShapeDtypeStruct((N_NODES, D_FEAT), jnp.float32),
    ]
    body = _u2_body
    if with_pool:
        out_specs.append(pl.BlockSpec((G_GRAPHS, D_FEAT), lambda i: (0, 0)))
        out_shape.append(jax.ShapeDtypeStruct((G_GRAPHS, D_FEAT), jnp.float32))
        body = _u2_pool_body
    return pl.pallas_call(
        body,
        grid=(N_NODES // R_BLK,),
        in_specs=in_specs,
        out_specs=out_specs,
        out_shape=out_shape,
    )(t2, sums, g, b, nimp, batch_f, segmax)


def kernel(x, edge_index, batch, node_imp,
           W1_0, b1_0, W2_0, b2_0, gamma_0, beta_0,
           W1_1, b1_1, W2_1, b2_1, gamma_1, beta_1,
           W1_2, b1_2, W2_2, b2_2, gamma_2, beta_2):
    params = [
        (W1_0, b1_0, W2_0, b2_0, gamma_0, beta_0),
        (W1_1, b1_1, W2_1, b2_1, gamma_1, beta_1),
        (W1_2, b1_2, W2_2, b2_2, gamma_2, beta_2),
    ]
    e = edge_index.shape[1]
    per_tile = -(-e // (NW * CHUNK * 8)) * (CHUNK * 8)
    e_pad = per_tile * NW
    pad = e_pad - e
    src = edge_index[0]
    dst = edge_index[1]
    if pad:
        src = jnp.concatenate([src, jnp.zeros((pad,), jnp.int32)])
        dst = jnp.concatenate([dst, jnp.full((pad,), N_NODES, jnp.int32)])
    src2 = src.reshape(e_pad // CHUNK, CHUNK)
    dst2 = dst.reshape(e_pad // CHUNK, CHUNK)

    batch_f = batch.astype(jnp.float32).reshape(N_NODES, 1)
    segmax = _segmax_call(node_imp, batch_f)

    h = x
    xs = []
    pool = None
    for i in range(N_LAYERS):
        w1, b1, w2, b2, g, b = params[i]
        agg = _aggregate(h, src2, dst2)
        t2, sums = _u1_call(agg, h, w1.reshape(D_FEAT, D_FEAT),
                            b1.reshape(1, D_FEAT), w2, b2.reshape(1, D_FEAT))
        outs = _u2_call(t2, sums, g.reshape(1, D_FEAT), b.reshape(1, D_FEAT),
                        node_imp, batch_f, segmax,
                        with_pool=(i == N_LAYERS - 1))
        h = outs[0]
        xs.append(outs[1])
        if i == N_LAYERS - 1:
            pool = outs[2]
    return pool, jnp.concatenate(xs, axis=1)


# speedup vs baseline: 3.0909x; 1.1145x over previous
import functools

import jax
import jax.numpy as jnp
from jax import lax
from jax.experimental import pallas as pl
from jax.experimental.pallas import tpu as pltpu
from jax.experimental.pallas import tpu_sc as plsc

N_NODES = 10000
D_FEAT = 128
G_GRAPHS = 128
N_LAYERS = 3
EPS_GIN = 0.1
EPS_BN = 1e-5

NW = 32
CHUNK = 128
N_PAD = 10240
R_BLK = 2000


@functools.lru_cache(maxsize=None)
def _sc_aggregate(kc: int):
    rows_per_tile = N_PAD // 16
    stage = 16

    @functools.partial(
        pl.kernel,
        out_type=jax.ShapeDtypeStruct((2, N_PAD, D_FEAT), jnp.float32),
        mesh=plsc.VectorSubcoreMesh(core_axis_name="c", subcore_axis_name="s"),
        scratch_types=[
            pltpu.VMEM((stage, CHUNK), jnp.int32),
            pltpu.VMEM((stage, CHUNK), jnp.int32),
            pltpu.VMEM((2, CHUNK, D_FEAT), jnp.float32),
            pltpu.VMEM_SHARED((N_PAD, D_FEAT), jnp.float32),
            pltpu.SemaphoreType.DMA,
            pltpu.SemaphoreType.DMA,
            pltpu.SemaphoreType.DMA,
            pltpu.SemaphoreType.DMA,
        ],
    )
    def agg(h_hbm, src_hbm, dst_hbm, out_hbm, sidx, didx, rows, acc,
            sg0, sg1, ss0, ss1):
        c = lax.axis_index("c")
        s = lax.axis_index("s")
        wid = c * 16 + s
        zero16 = jnp.zeros((16,), jnp.float32)
        semg = (sg0, sg1)
        sems = (ss0, ss1)

        def zrow(i, carry):
            for jj in range(D_FEAT // 16):
                rows[0, i, pl.ds(jj * 16, 16)] = zero16
            return carry

        lax.fori_loop(0, CHUNK, zrow, 0)
        base = s * rows_per_tile
        for k in range(rows_per_tile // CHUNK):
            pltpu.sync_copy(rows.at[0], acc.at[pl.ds(base + k * CHUNK, CHUNK)])
        plsc.subcore_barrier()

        def stage_body(t, carry):
            r0 = wid * kc + t * stage
            pltpu.sync_copy(src_hbm.at[pl.ds(r0, stage)], sidx)
            pltpu.sync_copy(dst_hbm.at[pl.ds(r0, stage)], didx)
            gh = [None, None]
            sh = [None, None]
            for j in range(stage + 1):
                b = j % 2
                if j < stage:
                    if sh[b] is not None:
                        sh[b].wait()
                    gh[b] = pltpu.async_copy(
                        h_hbm.at[sidx.at[j]], rows.at[b], semg[b])
                if j >= 1:
                    pb = (j - 1) % 2
                    gh[pb].wait()
                    sh[pb] = pltpu.async_copy(
                        rows.at[pb], acc.at[didx.at[j - 1]], sems[pb],
                        add=True)
            sh[0].wait()
            sh[1].wait()
            return carry

        lax.fori_loop(0, kc // stage, stage_body, 0)
        plsc.subcore_barrier()

        for k in range(rows_per_tile // CHUNK):
            r0 = base + k * CHUNK
            pltpu.sync_copy(acc.at[pl.ds(r0, CHUNK)], rows.at[0])
            pltpu.sync_copy(rows.at[0], out_hbm.at[c, pl.ds(r0, CHUNK)])

    return agg


def _aggregate(h, src2, dst2):
    return _sc_aggregate(src2.shape[0] // NW)(h, src2, dst2)


def _segmax_body(nimp_ref, batch_ref, out_ref):
    i = pl.program_id(0)
    gids = lax.broadcasted_iota(jnp.int32, (R_BLK, G_GRAPHS), 1).astype(jnp.float32)
    oh = batch_ref[...] == gids
    m = jnp.max(jnp.where(oh, nimp_ref[...], -jnp.inf), axis=0, keepdims=True)

    @pl.when(i == 0)
    def _():
        out_ref[...] = jnp.full_like(out_ref, -jnp.inf)

    out_ref[0:1, :] = jnp.maximum(out_ref[0:1, :], m)


def _segmax_call(nimp, batch_f):
    return pl.pallas_call(
        _segmax_body,
        grid=(N_NODES // R_BLK,),
        in_specs=[
            pl.BlockSpec((R_BLK, 1), lambda i: (i, 0)),
            pl.BlockSpec((R_BLK, 1), lambda i: (i, 0)),
        ],
        out_specs=pl.BlockSpec((8, G_GRAPHS), lambda i: (0, 0)),
        out_shape=jax.ShapeDtypeStruct((8, G_GRAPHS), jnp.float32),
    )(nimp, batch_f)


def _u1_body(agg_ref, h_ref, w1_ref, b1_ref, w2_ref, b2_ref, t2_ref, sums_ref):
    i = pl.program_id(0)
    t = agg_ref[0] + agg_ref[1] + (1.0 + EPS_GIN) * h_ref[...]
    t = jnp.maximum(
        jnp.dot(t, w1_ref[...], preferred_element_type=jnp.float32) + b1_ref[...],
        0.0,
    )
    t = jnp.dot(t, w2_ref[...], preferred_element_type=jnp.float32) + b2_ref[...]
    t = jnp.maximum(t, 0.0)
    t2_ref[...] = t

    @pl.when(i == 0)
    def _():
        sums_ref[...] = jnp.zeros_like(sums_ref)

    sums_ref[0:1, :] = sums_ref[0:1, :] + jnp.sum(t, axis=0, keepdims=True)
    sums_ref[1:2, :] = sums_ref[1:2, :] + jnp.sum(t * t, axis=0, keepdims=True)


def _u1_call(agg, h, w1, b1, w2, b2):
    return pl.pallas_call(
        _u1_body,
        grid=(N_NODES // R_BLK,),
        in_specs=[
            pl.BlockSpec((2, R_BLK, D_FEAT), lambda i: (0, i, 0)),
            pl.BlockSpec((R_BLK, D_FEAT), lambda i: (i, 0)),
            pl.BlockSpec((D_FEAT, D_FEAT), lambda i: (0, 0)),
            pl.BlockSpec((1, D_FEAT), lambda i: (0, 0)),
            pl.BlockSpec((D_FEAT, D_FEAT), lambda i: (0, 0)),
            pl.BlockSpec((1, D_FEAT), lambda i: (0, 0)),
        ],
        out_specs=[
            pl.BlockSpec((R_BLK, D_FEAT), lambda i: (i, 0)),
            pl.BlockSpec((8, D_FEAT), lambda i: (0, 0)),
        ],
        out_shape=[
            jax.ShapeDtypeStruct((N_NODES, D_FEAT), jnp.float32),
            jax.ShapeDtypeStruct((8, D_FEAT), jnp.float32),
        ],
    )(agg, h, w1, b1, w2, b2)


def _bn_ni(t2_ref, sums_ref, g_ref, b_ref, nimp_ref, batch_ref, segmax_ref):
    s0 = sums_ref[0:1, :]
    s1 = sums_ref[1:2, :]
    mean = s0 / N_NODES
    var = s1 / N_NODES - mean * mean
    scale = g_ref[...] / jnp.sqrt(var + EPS_BN)
    tn = (t2_ref[...] - mean) * scale + b_ref[...]
    gids = lax.broadcasted_iota(jnp.int32, (R_BLK, G_GRAPHS), 1).astype(jnp.float32)
    oh = batch_ref[...] == gids
    smax_row = jnp.sum(jnp.where(oh, segmax_ref[0:1, :], 0.0), axis=1, keepdims=True)
    ni = nimp_ref[...] / (smax_row * 10.0) + 0.9
    return tn, tn * ni, oh


def _u2_body(t2_ref, sums_ref, g_ref, b_ref, nimp_ref, batch_ref, segmax_ref,
             hn_ref, xs_ref):
    tn, xv, _ = _bn_ni(t2_ref, sums_ref, g_ref, b_ref, nimp_ref, batch_ref,
                       segmax_ref)
    hn_ref[...] = tn
    xs_ref[...] = xv


def _u2_pool_body(t2_ref, sums_ref, g_ref, b_ref, nimp_ref, batch_ref,
                  segmax_ref, hn_ref, xs_ref, pool_ref):
    i = pl.program_id(0)
    tn, xv, oh = _bn_ni(t2_ref, sums_ref, g_ref, b_ref, nimp_ref, batch_ref,
                        segmax_ref)
    hn_ref[...] = tn
    xs_ref[...] = xv

    @pl.when(i == 0)
    def _():
        pool_ref[...] = jnp.zeros_like(pool_ref)

    pool_ref[...] = pool_ref[...] + lax.dot_general(
        oh.astype(jnp.float32), xv, (((0,), (0,)), ((), ())),
        preferred_element_type=jnp.float32)


def _u2_call(t2, sums, g, b, nimp, batch_f, segmax, with_pool):
    in_specs = [
        pl.BlockSpec((R_BLK, D_FEAT), lambda i: (i, 0)),
        pl.BlockSpec((8, D_FEAT), lambda i: (0, 0)),
        pl.BlockSpec((1, D_FEAT), lambda i: (0, 0)),
        pl.BlockSpec((1, D_FEAT), lambda i: (0, 0)),
        pl.BlockSpec((R_BLK, 1), lambda i: (i, 0)),
        pl.BlockSpec((R_BLK, 1), lambda i: (i, 0)),
        pl.BlockSpec((8, G_GRAPHS), lambda i: (0, 0)),
    ]
    out_specs = [
        pl.BlockSpec((R_BLK, D_FEAT), lambda i: (i, 0)),
        pl.BlockSpec((R_BLK, D_FEAT), lambda i: (i, 0)),
    ]
    out_shape = [
        jax.ShapeDtypeStruct((N_NODES, D_FEAT), jnp.float32),
        jax.ShapeDtypeStruct((N_NODES, D_FEAT), jnp.float32),
    ]
    body = _u2_body
    if with_pool:
        out_specs.append(pl.BlockSpec((G_GRAPHS, D_FEAT), lambda i: (0, 0)))
        out_shape.append(jax.ShapeDtypeStruct((G_GRAPHS, D_FEAT), jnp.float32))
        body = _u2_pool_body
    return pl.pallas_call(
        body,
        grid=(N_NODES // R_BLK,),
        in_specs=in_specs,
        out_specs=out_specs,
        out_shape=out_shape,
    )(t2, sums, g, b, nimp, batch_f, segmax)


def kernel(x, edge_index, batch, node_imp,
           W1_0, b1_0, W2_0, b2_0, gamma_0, beta_0,
           W1_1, b1_1, W2_1, b2_1, gamma_1, beta_1,
           W1_2, b1_2, W2_2, b2_2, gamma_2, beta_2):
    params = [
        (W1_0, b1_0, W2_0, b2_0, gamma_0, beta_0),
        (W1_1, b1_1, W2_1, b2_1, gamma_1, beta_1),
        (W1_2, b1_2, W2_2, b2_2, gamma_2, beta_2),
    ]
    e = edge_index.shape[1]
    per_tile = -(-e // (NW * CHUNK * 8)) * (CHUNK * 8)
    e_pad = per_tile * NW
    pad = e_pad - e
    src = edge_index[0]
    dst = edge_index[1]
    if pad:
        src = jnp.concatenate([src, jnp.zeros((pad,), jnp.int32)])
        dst = jnp.concatenate([dst, jnp.full((pad,), N_NODES, jnp.int32)])
    src2 = src.reshape(e_pad // CHUNK, CHUNK)
    dst2 = dst.reshape(e_pad // CHUNK, CHUNK)

    batch_f = batch.astype(jnp.float32).reshape(N_NODES, 1)
    segmax = _segmax_call(node_imp, batch_f)

    h = x
    xs = []
    pool = None
    for i in range(N_LAYERS):
        w1, b1, w2, b2, g, b = params[i]
        agg = _aggregate(h, src2, dst2)
        t2, sums = _u1_call(agg, h, w1.reshape(D_FEAT, D_FEAT),
                            b1.reshape(1, D_FEAT), w2, b2.reshape(1, D_FEAT))
        outs = _u2_call(t2, sums, g.reshape(1, D_FEAT), b.reshape(1, D_FEAT),
                        node_imp, batch_f, segmax,
                        with_pool=(i == N_LAYERS - 1))
        h = outs[0]
        xs.append(outs[1])
        if i == N_LAYERS - 1:
            pool = outs[2]
    return pool, jnp.concatenate(xs, axis=1)
